# baseline (speedup 1.0000x reference)
import functools

import jax
import jax.numpy as jnp
from jax import lax
from jax.experimental import pallas as pl
from jax.experimental.pallas import tpu as pltpu
from jax.experimental.pallas import tpu_sc as plsc

NUM_CORES = 2
NUM_SUBCORES = 16
LANES = 16
NW = NUM_CORES * NUM_SUBCORES

EMBED_DIM = 64
BATCH = 16384
B_PER_W = BATCH // NW
CHUNK = LANES
N_CHUNKS = B_PER_W // CHUNK


def _rsqrt(x):
    i = plsc.bitcast(x, jnp.int32)
    y = plsc.bitcast(jnp.int32(0x5F3759DF) - (i >> 1), jnp.float32)
    for _ in range(3):
        y = y * (jnp.float32(1.5) - jnp.float32(0.5) * x * y * y)
    return y


def _sc_body(table_hbm, nodes_hbm, out_hbm, idx_v, rows_v, t_v, gsem):
    wid = lax.axis_index("s") * NUM_CORES + lax.axis_index("c")
    base = wid * B_PER_W

    pltpu.sync_copy(nodes_hbm.at[wid], idx_v)

    for c in range(N_CHUNKS):
        pltpu.async_copy(table_hbm.at[idx_v.at[c]],
                         rows_v.at[pl.ds(c * CHUNK, CHUNK)], gsem)
    for c in range(N_CHUNKS):
        pltpu.make_async_copy(table_hbm.at[idx_v.at[c]],
                              rows_v.at[pl.ds(c * CHUNK, CHUNK)], gsem).wait()

    lane = lax.broadcasted_iota(jnp.int32, (LANES,), 0)

    def chunk_body(c):
        row = c * CHUNK + lane
        acc = jnp.zeros((LANES,), jnp.float32)
        for d in range(EMBED_DIM):
            v = plsc.load_gather(rows_v, [row, jnp.full((LANES,), d, jnp.int32)])
            acc = acc + v * v
            t_v[d, pl.ds(c * CHUNK, CHUNK)] = v
        r = _rsqrt(acc)
        for d in range(EMBED_DIM):
            t_v[d, pl.ds(c * CHUNK, CHUNK)] = t_v[d, pl.ds(c * CHUNK, CHUNK)] * r

    pl.loop(0, N_CHUNKS)(chunk_body)

    pltpu.sync_copy(t_v, out_hbm.at[:, pl.ds(base, B_PER_W)])


@jax.jit
def _encode(nodes_r, table):
    mesh = plsc.VectorSubcoreMesh(core_axis_name="c", subcore_axis_name="s")
    return pl.kernel(
        _sc_body,
        out_type=jax.ShapeDtypeStruct((EMBED_DIM, BATCH), jnp.float32),
        mesh=mesh,
        compiler_params=pltpu.CompilerParams(needs_layout_passes=False,
                                             use_tc_tiling_on_sc=False),
        scratch_types=[
            pltpu.VMEM((N_CHUNKS, CHUNK), jnp.int32),
            pltpu.VMEM((B_PER_W, EMBED_DIM), jnp.float32),
            pltpu.VMEM((EMBED_DIM, B_PER_W), jnp.float32),
            pltpu.SemaphoreType.DMA,
        ],
    )(table, nodes_r)


def kernel(nodes, table):
    nodes_r = nodes.astype(jnp.int32).reshape(NW, N_CHUNKS, CHUNK)
    return _encode(nodes_r, table)


# speedup vs baseline: 1.0028x; 1.0028x over previous
import jax
import jax.numpy as jnp
from jax import lax
from jax.experimental import pallas as pl
from jax.experimental.pallas import tpu as pltpu
from jax.experimental.pallas import tpu_sc as plsc

NUM_CORES = 2
NUM_SUBCORES = 16
LANES = 16
NW = NUM_CORES * NUM_SUBCORES

EMBED_DIM = 64
BATCH = 16384
B_PER_W = BATCH // NW
CHUNK = LANES
N_CHUNKS = B_PER_W // CHUNK
PAIR_DIM = 2 * EMBED_DIM


def _rsqrt(x):
    i = plsc.bitcast(x, jnp.int32)
    y = plsc.bitcast(jnp.int32(0x5F3759DF) - (i >> 1), jnp.float32)
    for _ in range(3):
        y = y * (jnp.float32(1.5) - jnp.float32(0.5) * x * y * y)
    return y


def _sc_body(table_hbm, nodes_hbm, out_hbm, idx_v, rows_v, t_v, gsem):
    wid = lax.axis_index("s") * NUM_CORES + lax.axis_index("c")
    base = wid * B_PER_W

    pltpu.sync_copy(nodes_hbm.at[wid], idx_v)

    for c in range(N_CHUNKS):
        hi = idx_v[c, :] >> 1
        pltpu.async_copy(table_hbm.at[hi],
                         rows_v.at[pl.ds(c * CHUNK, CHUNK)], gsem)
    pltpu.make_async_copy(
        table_hbm.at[jnp.zeros((B_PER_W,), jnp.int32)], rows_v, gsem
    ).wait()

    lane = lax.broadcasted_iota(jnp.int32, (LANES,), 0)

    def chunk_body(c):
        row = c * CHUNK + lane
        par = (idx_v[c, :] & 1) * EMBED_DIM
        acc = jnp.zeros((LANES,), jnp.float32)
        for d in range(EMBED_DIM):
            v = plsc.load_gather(rows_v, [row, par + d])
            acc = acc + v * v
            t_v[d, pl.ds(c * CHUNK, CHUNK)] = v
        r = _rsqrt(acc)
        for d in range(EMBED_DIM):
            t_v[d, pl.ds(c * CHUNK, CHUNK)] = t_v[d, pl.ds(c * CHUNK, CHUNK)] * r

    pl.loop(0, N_CHUNKS)(chunk_body)

    pltpu.sync_copy(t_v, out_hbm.at[:, pl.ds(base, B_PER_W)])


@jax.jit
def _encode(nodes, table):
    nodes_r = nodes.astype(jnp.int32).reshape(NW, N_CHUNKS, CHUNK)
    table_p = table.reshape(table.shape[0] // 2, PAIR_DIM)
    mesh = plsc.VectorSubcoreMesh(core_axis_name="c", subcore_axis_name="s")
    return pl.kernel(
        _sc_body,
        out_type=jax.ShapeDtypeStruct((EMBED_DIM, BATCH), jnp.float32),
        mesh=mesh,
        compiler_params=pltpu.CompilerParams(needs_layout_passes=False),
        scratch_types=[
            pltpu.VMEM((N_CHUNKS, CHUNK), jnp.int32),
            pltpu.VMEM((B_PER_W, PAIR_DIM), jnp.float32),
            pltpu.VMEM((EMBED_DIM, B_PER_W), jnp.float32),
            pltpu.SemaphoreType.DMA,
        ],
    )(table_p, nodes_r)


def kernel(nodes, table):
    return _encode(nodes, table)
